# baseline (speedup 1.0000x reference)
import functools

import jax
import jax.numpy as jnp
from jax import lax
from jax.experimental import pallas as pl
from jax.experimental.pallas import tpu as pltpu
from jax.experimental.pallas import tpu_sc as plsc

NC = 2
NS = 16
L = 16
NW = NC * NS

BATCH = 4
SEQ = 8192
HIDDEN = 128
HCH = HIDDEN // L
P = SEQ // NW
CHUNK = 128
NCHUNK = (BATCH * P) // CHUNK
NBUF = 3
UNROLL = 4
EPS = 1e-12

_SHUF_DN = lax.GatherDimensionNumbers(
    offset_dims=(), collapsed_slice_dims=(0,), start_index_map=(0,))


def _lane_sum(x):
    lanes = lax.iota(jnp.int32, L)
    for k in (8, 4, 2, 1):
        idx = (lanes ^ k).reshape(L, 1)
        x = x + lax.gather(x, idx, _SHUF_DN, (1,),
                           mode=lax.GatherScatterMode.PROMISE_IN_BOUNDS)
    return x


def _token_body(wordbuf, posbuf, gv, bv, t, tp):
    xs = []
    for h in range(HCH):
        w = wordbuf[t, pl.ds(L * h, L)]
        p = posbuf[tp, pl.ds(L * h, L)]
        xs.append(w + p)
    s = xs[0]
    sq = xs[0] * xs[0]
    for h in range(1, HCH):
        s = s + xs[h]
        sq = sq + xs[h] * xs[h]
    sv = _lane_sum(s)
    qv = _lane_sum(sq)
    meanv = sv * (1.0 / HIDDEN)
    varv = qv * (1.0 / HIDDEN) - meanv * meanv
    rv = varv + EPS
    iv = lax.bitcast_convert_type(rv, jnp.int32)
    iv = jnp.int32(0x5F3759DF) - lax.shift_right_arithmetic(iv, 1)
    y = lax.bitcast_convert_type(iv, jnp.float32)
    y = y * (1.5 - 0.5 * rv * y * y)
    y = y * (1.5 - 0.5 * rv * y * y)
    mv = meanv * y
    for h in range(HCH):
        o = xs[h] * y - mv
        wordbuf[t, pl.ds(L * h, L)] = o * gv[h] + bv[h]


def _sc_embed_ln(ids2d, word_emb, pos_emb, type_emb, gamma, beta):
    mesh = plsc.VectorSubcoreMesh(core_axis_name="c", subcore_axis_name="s")

    @functools.partial(
        pl.kernel,
        mesh=mesh,
        out_type=jax.ShapeDtypeStruct((BATCH * SEQ, HIDDEN), jnp.float32),
        scratch_types=[
            pltpu.VMEM((P, HIDDEN), jnp.float32),
            pltpu.VMEM((NBUF, CHUNK, HIDDEN), jnp.float32),
            pltpu.VMEM((NCHUNK, CHUNK), jnp.int32),
            pltpu.VMEM((HIDDEN,), jnp.float32),
            pltpu.VMEM((HIDDEN,), jnp.float32),
            pltpu.VMEM((1, HIDDEN), jnp.float32),
            pltpu.SemaphoreType.DMA,
            pltpu.SemaphoreType.DMA((NBUF,)),
            pltpu.SemaphoreType.DMA((NBUF,)),
        ],
    )
    def k(ids_hbm, word_hbm, pos_hbm, type_hbm, gamma_hbm, beta_hbm,
          out_hbm, posbuf, wordbuf, idxbuf, gbuf, bbuf, tbuf,
          ssem, gsem, osem):
        wid = lax.axis_index("s") * NC + lax.axis_index("c")
        pos_base = wid * P
        idx_rows = [None] * NCHUNK

        setup = [
            pltpu.async_copy(pos_hbm.at[pl.ds(pos_base, P)], posbuf, ssem),
            pltpu.async_copy(type_hbm.at[pl.ds(0, 1)], tbuf, ssem),
            pltpu.async_copy(gamma_hbm, gbuf, ssem),
            pltpu.async_copy(beta_hbm, bbuf, ssem),
        ]
        for b in range(BATCH):
            setup.append(pltpu.async_copy(
                ids_hbm.at[pl.ds(64 * b + 2 * wid, 2)],
                idxbuf.at[pl.ds(2 * b, 2)], ssem))
        for cp in setup:
            cp.wait()

        def gather(c):
            return pltpu.async_copy(
                word_hbm.at[idxbuf.at[c]], wordbuf.at[c % NBUF],
                gsem.at[c % NBUF])

        gathers = [gather(0), gather(1)]

        tv = [tbuf[0, pl.ds(L * h, L)] for h in range(HCH)]
        gv = [gbuf[pl.ds(L * h, L)] for h in range(HCH)]
        bv = [bbuf[pl.ds(L * h, L)] for h in range(HCH)]

        def add_type(t, _):
            for h in range(HCH):
                posbuf[t, pl.ds(L * h, L)] = posbuf[t, pl.ds(L * h, L)] + tv[h]
            return _
        lax.fori_loop(0, P, add_type, None)

        writebacks = [None] * NCHUNK
        for c in range(NCHUNK):
            b, j = c // 2, c % 2
            buf = c % NBUF
            gathers[c].wait()
            wb = wordbuf.at[buf]

            def tok(i, _, wb=wb, j=j):
                for u in range(UNROLL):
                    t = i * UNROLL + u
                    _token_body(wb, posbuf, gv, bv, t, j * CHUNK + t)
                return _
            lax.fori_loop(0, CHUNK // UNROLL, tok, None)

            chunk_base = b * SEQ + pos_base + j * CHUNK
            writebacks[c] = pltpu.async_copy(
                wb, out_hbm.at[pl.ds(chunk_base, CHUNK)], osem.at[buf])
            if c + 2 < NCHUNK:
                if c - 1 >= 0:
                    writebacks[c - 1].wait()
                gathers.append(gather(c + 2))
        for c in range(NCHUNK - NBUF, NCHUNK):
            if writebacks[c] is not None and c >= NCHUNK - NBUF:
                writebacks[c].wait()

    return k(ids2d, word_emb, pos_emb, type_emb, gamma, beta)


def kernel(input_ids, word_emb, pos_emb, type_emb, gamma, beta):
    ids2d = input_ids.reshape(-1, CHUNK).astype(jnp.int32)
    out = _sc_embed_ln(ids2d, word_emb, pos_emb, type_emb, gamma, beta)
    return out.reshape(BATCH, SEQ, HIDDEN)


# speedup vs baseline: 1.0186x; 1.0186x over previous
import functools

import jax
import jax.numpy as jnp
from jax import lax
from jax.experimental import pallas as pl
from jax.experimental.pallas import tpu as pltpu
from jax.experimental.pallas import tpu_sc as plsc

NC = 2
NS = 16
L = 16
NW = NC * NS

BATCH = 4
SEQ = 8192
HIDDEN = 128
HCH = HIDDEN // L
P = SEQ // NW
CHUNK = 128
NCHUNK = (BATCH * P) // CHUNK
NBUF = 3
UNROLL = 4
EPS = 1e-12

_SHUF_DN = lax.GatherDimensionNumbers(
    offset_dims=(), collapsed_slice_dims=(0,), start_index_map=(0,))


def _lane_sum(x):
    lanes = lax.iota(jnp.int32, L)
    for k in (8, 4, 2, 1):
        idx = (lanes ^ k).reshape(L, 1)
        x = x + lax.gather(x, idx, _SHUF_DN, (1,),
                           mode=lax.GatherScatterMode.PROMISE_IN_BOUNDS)
    return x


def _token_body(wordbuf, posbuf, gv, bv, t, tp):
    xs = []
    for h in range(HCH):
        w = wordbuf[t, pl.ds(L * h, L)]
        p = posbuf[tp, pl.ds(L * h, L)]
        xs.append(w + p)
    s = xs[0]
    sq = xs[0] * xs[0]
    for h in range(1, HCH):
        s = s + xs[h]
        sq = sq + xs[h] * xs[h]
    sv = _lane_sum(s)
    qv = _lane_sum(sq)
    meanv = sv * (1.0 / HIDDEN)
    varv = qv * (1.0 / HIDDEN) - meanv * meanv
    rv = varv + EPS
    iv = lax.bitcast_convert_type(rv, jnp.int32)
    iv = jnp.int32(0x5F3759DF) - lax.shift_right_arithmetic(iv, 1)
    y = lax.bitcast_convert_type(iv, jnp.float32)
    y = y * (1.5 - 0.5 * rv * y * y)
    y = y * (1.5 - 0.5 * rv * y * y)
    mv = meanv * y
    for h in range(HCH):
        o = xs[h] * y - mv
        wordbuf[t, pl.ds(L * h, L)] = o * gv[h] + bv[h]


def _sc_embed_ln(ids2d, word_emb, pos_emb, type_emb, gamma, beta):
    mesh = plsc.VectorSubcoreMesh(core_axis_name="c", subcore_axis_name="s")

    @functools.partial(
        pl.kernel,
        mesh=mesh,
        out_type=jax.ShapeDtypeStruct((BATCH * SEQ, HIDDEN), jnp.float32),
        scratch_types=[
            pltpu.VMEM((P, HIDDEN), jnp.float32),
            pltpu.VMEM((NBUF, CHUNK, HIDDEN), jnp.float32),
            pltpu.VMEM((NCHUNK, CHUNK), jnp.int32),
            pltpu.VMEM((HIDDEN,), jnp.float32),
            pltpu.VMEM((HIDDEN,), jnp.float32),
            pltpu.VMEM((1, HIDDEN), jnp.float32),
            pltpu.SemaphoreType.DMA,
            pltpu.SemaphoreType.DMA((NBUF,)),
            pltpu.SemaphoreType.DMA((NBUF,)),
        ],
    )
    def k(ids_hbm, word_hbm, pos_hbm, type_hbm, gamma_hbm, beta_hbm,
          out_hbm, posbuf, wordbuf, idxbuf, gbuf, bbuf, tbuf,
          ssem, gsem, osem):
        wid = lax.axis_index("s") * NC + lax.axis_index("c")
        pos_base = wid * P
        idx_rows = [None] * NCHUNK

        setup = [
            pltpu.async_copy(pos_hbm.at[pl.ds(pos_base, P)], posbuf, ssem),
            pltpu.async_copy(type_hbm.at[pl.ds(0, 1)], tbuf, ssem),
            pltpu.async_copy(gamma_hbm, gbuf, ssem),
            pltpu.async_copy(beta_hbm, bbuf, ssem),
        ]
        for b in range(BATCH):
            setup.append(pltpu.async_copy(
                ids_hbm.at[pl.ds(64 * b + 2 * wid, 2)],
                idxbuf.at[pl.ds(2 * b, 2)], ssem))
        for cp in setup:
            cp.wait()

        def gather(c):
            return pltpu.async_copy(
                word_hbm.at[idxbuf.at[c]], wordbuf.at[c % NBUF],
                gsem.at[c % NBUF])

        gathers = [gather(0), gather(1)]

        tv = [tbuf[0, pl.ds(L * h, L)] for h in range(HCH)]
        gv = [gbuf[pl.ds(L * h, L)] for h in range(HCH)]
        bv = [bbuf[pl.ds(L * h, L)] for h in range(HCH)]

        @plsc.parallel_loop(0, P, unroll=4)
        def add_type(t):
            for h in range(HCH):
                posbuf[t, pl.ds(L * h, L)] = posbuf[t, pl.ds(L * h, L)] + tv[h]

        writebacks = [None] * NCHUNK
        for c in range(NCHUNK):
            b, j = c // 2, c % 2
            buf = c % NBUF
            gathers[c].wait()
            wb = wordbuf.at[buf]

            @plsc.parallel_loop(0, CHUNK, unroll=UNROLL)
            def tok(t, wb=wb, j=j):
                _token_body(wb, posbuf, gv, bv, t, j * CHUNK + t)

            chunk_base = b * SEQ + pos_base + j * CHUNK
            writebacks[c] = pltpu.async_copy(
                wb, out_hbm.at[pl.ds(chunk_base, CHUNK)], osem.at[buf])
            if c + 2 < NCHUNK:
                if c - 1 >= 0:
                    writebacks[c - 1].wait()
                gathers.append(gather(c + 2))
        for c in range(NCHUNK - NBUF, NCHUNK):
            if writebacks[c] is not None and c >= NCHUNK - NBUF:
                writebacks[c].wait()

    return k(ids2d, word_emb, pos_emb, type_emb, gamma, beta)


def kernel(input_ids, word_emb, pos_emb, type_emb, gamma, beta):
    ids2d = input_ids.reshape(-1, CHUNK).astype(jnp.int32)
    out = _sc_embed_ln(ids2d, word_emb, pos_emb, type_emb, gamma, beta)
    return out.reshape(BATCH, SEQ, HIDDEN)
